# Initial kernel scaffold; baseline (speedup 1.0000x reference)
import functools

import jax
import jax.numpy as jnp
from jax import lax
from jax.experimental import pallas as pl
from jax.experimental.pallas import tpu as pltpu
from jax.experimental.pallas import tpu_sc as plsc

N = 50000
E = 150000
NREL = 4
H = 128
CLS = 16

NC = 2
NS = 16
NW = NC * NS
LANES = 16

NPAD = 50176
DUST = 128
CH = 128
ET = 4736
NCH = ET // CH
EPAD = NW * ET
ESC = EPAD // NS
DEGROWS = NPAD + DUST


def _prep_kernel(dst_hbm, scale_hbm, deg0, deg1, deg2, deg3, inv_vm, buf_vm,
                 idx_vm, val_vm, sem):
  c = lax.axis_index("c")
  s = lax.axis_index("s")
  w = s * NC + c
  degs = [deg0, deg1, deg2, deg3]
  zeros16 = jnp.zeros((LANES,), jnp.float32)
  ones16 = jnp.ones((LANES,), jnp.float32)

  def zbuf(i, _):
    buf_vm[pl.ds(i * LANES, LANES)] = zeros16
    return 0
  SL = DEGROWS // NS
  lax.fori_loop(0, SL // LANES, zbuf, 0)
  for r in range(NREL):
    pltpu.sync_copy(buf_vm.at[pl.ds(0, SL)], degs[r].at[pl.ds(s * SL, SL)])
  plsc.subcore_barrier()

  def ofill(i, _):
    val_vm[pl.ds(i * LANES, LANES)] = ones16
    return 0
  lax.fori_loop(0, CH // LANES, ofill, 0)
  for r in range(NREL):
    def cnt_body(ci, _):
      off = s * ESC + ci * CH
      pltpu.sync_copy(dst_hbm.at[r, pl.ds(off, CH)], idx_vm)
      pltpu.sync_copy(val_vm, degs[r].at[idx_vm], add=True)
      return 0
    lax.fori_loop(0, ESC // CH, cnt_body, 0)
  plsc.subcore_barrier()

  for r in range(NREL):
    pltpu.sync_copy(degs[r].at[pl.ds(s * SL, SL)], buf_vm.at[pl.ds(0, SL)])
    def invb(i, _):
      v = buf_vm[pl.ds(i * LANES, LANES)]
      buf_vm[pl.ds(i * LANES, LANES)] = 1.0 / jnp.maximum(v, 1.0)
      return 0
    lax.fori_loop(0, SL // LANES, invb, 0)
    pltpu.sync_copy(buf_vm.at[pl.ds(0, SL)], degs[r].at[pl.ds(s * SL, SL)])
  plsc.subcore_barrier()

  for r in range(NREL):
    pltpu.sync_copy(degs[r], inv_vm)
    def sc_body(ci, _):
      off = w * ET + ci * CH
      pltpu.sync_copy(dst_hbm.at[r, pl.ds(off, CH)], idx_vm)
      for k in range(CH // LANES):
        dv = idx_vm[pl.ds(k * LANES, LANES)]
        val_vm[pl.ds(k * LANES, LANES)] = plsc.load_gather(inv_vm, [dv])
      pltpu.sync_copy(val_vm, scale_hbm.at[r, pl.ds(off, CH)])
      return 0
    lax.fori_loop(0, NCH, sc_body, 0)


def _make_prep():
  mesh = plsc.VectorSubcoreMesh(core_axis_name="c", subcore_axis_name="s")
  scratch = [
      pltpu.VMEM_SHARED((DEGROWS,), jnp.float32),
      pltpu.VMEM_SHARED((DEGROWS,), jnp.float32),
      pltpu.VMEM_SHARED((DEGROWS,), jnp.float32),
      pltpu.VMEM_SHARED((DEGROWS,), jnp.float32),
      pltpu.VMEM((DEGROWS,), jnp.float32),
      pltpu.VMEM((DEGROWS // NS,), jnp.float32),
      pltpu.VMEM((CH,), jnp.int32),
      pltpu.VMEM((CH,), jnp.float32),
      pltpu.SemaphoreType.DMA,
  ]
  return pl.kernel(
      _prep_kernel,
      out_type=jax.ShapeDtypeStruct((NREL, EPAD), jnp.float32),
      mesh=mesh,
      scratch_types=scratch,
  )


def _agg_kernel(NB, D, t0, t1, t2, t3, src_hbm, dst_hbm, scale_hbm, out_hbm,
                acc, rows, dstv, locv, srcv, sclv, sem):
  c = lax.axis_index("c")
  s = lax.axis_index("s")
  w = s * NC + c
  tables = [t0, t1, t2, t3]
  BSZ = NPAD // NB
  ACC = BSZ + DUST
  ZR = ACC // NS
  FLR = BSZ // NS
  iota = lax.iota(jnp.int32, LANES)
  zeros16 = jnp.zeros((LANES,), jnp.float32)

  for b in range(NB):
    lo = b * BSZ
    def zrow(i, _):
      for j in range(D // LANES):
        rows[i, pl.ds(j * LANES, LANES)] = zeros16
      return 0
    lax.fori_loop(0, CH, zrow, 0)
    nfull, rem = ZR // CH, ZR % CH
    for k in range(nfull):
      pltpu.sync_copy(rows, acc.at[pl.ds(s * ZR + k * CH, CH)])
    if rem:
      pltpu.sync_copy(rows.at[pl.ds(0, rem)],
                      acc.at[pl.ds(s * ZR + nfull * CH, rem)])
    plsc.subcore_barrier()

    for r in range(NREL):
      table = tables[r]
      def chunk(ci, _):
        off = w * ET + ci * CH
        pltpu.sync_copy(dst_hbm.at[r, pl.ds(off, CH)], dstv)
        pltpu.sync_copy(src_hbm.at[r, pl.ds(off, CH)], srcv)
        pltpu.sync_copy(scale_hbm.at[r, pl.ds(off, CH)], sclv)
        for k in range(CH // LANES):
          dv = dstv[pl.ds(k * LANES, LANES)]
          inb = (dv >= lo) & (dv < lo + BSZ)
          locv[pl.ds(k * LANES, LANES)] = jnp.where(
              inb, dv - lo, BSZ + k * LANES + iota)
        pltpu.async_copy(table.at[srcv], rows, sem).wait()
        def srow(i, _):
          sv = jnp.full((LANES,), sclv[i], jnp.float32)
          for j in range(D // LANES):
            rows[i, pl.ds(j * LANES, LANES)] = (
                rows[i, pl.ds(j * LANES, LANES)] * sv)
          return 0
        lax.fori_loop(0, CH, srow, 0)
        pltpu.sync_copy(rows, acc.at[locv], add=True)
        return 0
      lax.fori_loop(0, NCH, chunk, 0)
    plsc.subcore_barrier()

    pltpu.sync_copy(acc.at[pl.ds(s * FLR, FLR)],
                    out_hbm.at[c, pl.ds(lo + s * FLR, FLR)])
    plsc.subcore_barrier()


def _make_agg(NB, D):
  mesh = plsc.VectorSubcoreMesh(core_axis_name="c", subcore_axis_name="s")
  BSZ = NPAD // NB
  scratch = [
      pltpu.VMEM_SHARED((BSZ + DUST, D), jnp.float32),
      pltpu.VMEM((CH, D), jnp.float32),
      pltpu.VMEM((CH,), jnp.int32),
      pltpu.VMEM((CH,), jnp.int32),
      pltpu.VMEM((CH,), jnp.int32),
      pltpu.VMEM((CH,), jnp.float32),
      pltpu.SemaphoreType.DMA,
  ]
  return pl.kernel(
      functools.partial(_agg_kernel, NB, D),
      out_type=jax.ShapeDtypeStruct((NC, NPAD, D), jnp.float32),
      mesh=mesh,
      scratch_types=scratch,
  )


def _tc_transform(p0, p1, bias, wc, basis):
  Dout = basis.shape[2]
  R = 512
  G = NPAD // R

  def body(p0_ref, p1_ref, b_ref, wc_ref, ba_ref, out_ref):
    x = p0_ref[...] + p1_ref[...] + b_ref[...]
    h = jnp.maximum(x, 0.0)
    wcv = wc_ref[...]
    for r in range(NREL):
      wr = wcv[r, 0] * ba_ref[0] + wcv[r, 1] * ba_ref[1]
      out_ref[r] = jnp.dot(h, wr, preferred_element_type=jnp.float32)

  return pl.pallas_call(
      body,
      grid=(G,),
      in_specs=[
          pl.BlockSpec((R, H), lambda i: (i, 0)),
          pl.BlockSpec((R, H), lambda i: (i, 0)),
          pl.BlockSpec((1, H), lambda i: (0, 0)),
          pl.BlockSpec((NREL, 2), lambda i: (0, 0)),
          pl.BlockSpec((2, H, Dout), lambda i: (0, 0, 0)),
      ],
      out_specs=pl.BlockSpec((NREL, R, Dout), lambda i: (0, i, 0)),
      out_shape=jax.ShapeDtypeStruct((NREL, NPAD, Dout), jnp.float32),
  )(p0, p1, bias.reshape(1, H), wc, basis)


def _tc_final(p0, p1, bias2):
  R = 400
  G = N // R

  def body(p0_ref, p1_ref, b_ref, out_ref):
    out_ref[...] = p0_ref[...] + p1_ref[...] + b_ref[...]

  return pl.pallas_call(
      body,
      grid=(G,),
      in_specs=[
          pl.BlockSpec((R, CLS), lambda i: (i, 0)),
          pl.BlockSpec((R, CLS), lambda i: (i, 0)),
          pl.BlockSpec((1, CLS), lambda i: (0, 0)),
      ],
      out_specs=pl.BlockSpec((R, CLS), lambda i: (i, 0)),
      out_shape=jax.ShapeDtypeStruct((N, CLS), jnp.float32),
  )(p0, p1, bias2.reshape(1, CLS))


def kernel(edge_index, embeds, embed_bias, weight1, w_comp1, bias1, weight2,
           w_comp2, bias2):
  src = edge_index[:, 0, :]
  dst = edge_index[:, 1, :]
  src_p = jnp.pad(src, ((0, 0), (0, EPAD - E)), constant_values=0)
  dst_p = jnp.pad(dst, ((0, 0), (0, EPAD - E)), constant_values=NPAD)

  scale = _make_prep()(dst_p)

  agg128 = _make_agg(4, H)
  agg16 = _make_agg(1, CLS)

  pA = agg128(embeds[0], embeds[1], embeds[2], embeds[3],
              src_p, dst_p, scale)
  y1 = _tc_transform(pA[0], pA[1], embed_bias, w_comp1, weight1)
  pB = agg128(y1[0], y1[1], y1[2], y1[3], src_p, dst_p, scale)
  y2 = _tc_transform(pB[0], pB[1], bias1, w_comp2, weight2)
  pC = agg16(y2[0], y2[1], y2[2], y2[3], src_p, dst_p, scale)
  return _tc_final(pC[0], pC[1], bias2)


# speedup vs baseline: 1.2821x; 1.2821x over previous
import functools

import jax
import jax.numpy as jnp
from jax import lax
from jax.experimental import pallas as pl
from jax.experimental.pallas import tpu as pltpu
from jax.experimental.pallas import tpu_sc as plsc

N = 50000
E = 150000
NREL = 4
H = 128
CLS = 16

NC = 2
NS = 16
NW = NC * NS
LANES = 16

NPAD = 50176
DUST = 128
CH = 128
ET = 4736
NCH = ET // CH
EPAD = NW * ET
ESC = EPAD // NS
DEGROWS = 51200
SL = DEGROWS // NS


def _prep_kernel(dst_hbm, scale_hbm, inv_hbm, deg0, deg1, deg2, deg3, buf_vm,
                 idx_vm, gidx_vm, val_vm, sem):
  c = lax.axis_index("c")
  s = lax.axis_index("s")
  w = s * NC + c
  degs = [deg0, deg1, deg2, deg3]
  zeros16 = jnp.zeros((LANES,), jnp.float32)
  ones16 = jnp.ones((LANES,), jnp.float32)

  def zbuf(i, _):
    buf_vm[pl.ds(i * LANES, LANES)] = zeros16
    return 0
  lax.fori_loop(0, SL // LANES, zbuf, 0)
  for r in range(NREL):
    pltpu.sync_copy(buf_vm.at[pl.ds(0, SL)], degs[r].at[pl.ds(s * SL, SL)])
  plsc.subcore_barrier()

  def ofill(i, _):
    val_vm[pl.ds(i * LANES, LANES)] = ones16
    return 0
  lax.fori_loop(0, CH // LANES, ofill, 0)
  for r in range(NREL):
    def cnt_body(ci, _):
      off = r * EPAD + s * ESC + ci * CH
      pltpu.sync_copy(dst_hbm.at[pl.ds(off, CH)], idx_vm)
      pltpu.sync_copy(val_vm, degs[r].at[idx_vm], add=True)
      return 0
    lax.fori_loop(0, ESC // CH, cnt_body, 0)
  plsc.subcore_barrier()

  for r in range(NREL):
    pltpu.sync_copy(degs[r].at[pl.ds(s * SL, SL)], buf_vm.at[pl.ds(0, SL)])
    def invb(i, _):
      v = buf_vm[pl.ds(i * LANES, LANES)]
      buf_vm[pl.ds(i * LANES, LANES)] = 1.0 / jnp.maximum(v, 1.0)
      return 0
    lax.fori_loop(0, SL // LANES, invb, 0)
    pltpu.sync_copy(
        buf_vm.at[pl.ds(0, SL)],
        inv_hbm.at[pl.ds(c * NREL * DEGROWS + r * DEGROWS + s * SL, SL)])
  plsc.subcore_barrier()

  for r in range(NREL):
    base = c * (NREL * DEGROWS) + r * DEGROWS
    def sc_body(ci, _):
      off = r * EPAD + w * ET + ci * CH
      pltpu.sync_copy(dst_hbm.at[pl.ds(off, CH)], idx_vm)
      for k in range(CH // LANES):
        gidx_vm[pl.ds(k * LANES, LANES)] = (
            idx_vm[pl.ds(k * LANES, LANES)] + base)
      pltpu.async_copy(inv_hbm.at[gidx_vm], val_vm, sem).wait()
      pltpu.sync_copy(val_vm, scale_hbm.at[pl.ds(off, CH)])
      return 0
    lax.fori_loop(0, NCH, sc_body, 0)


def _make_prep():
  mesh = plsc.VectorSubcoreMesh(core_axis_name="c", subcore_axis_name="s")
  scratch = [
      pltpu.VMEM_SHARED((DEGROWS,), jnp.float32),
      pltpu.VMEM_SHARED((DEGROWS,), jnp.float32),
      pltpu.VMEM_SHARED((DEGROWS,), jnp.float32),
      pltpu.VMEM_SHARED((DEGROWS,), jnp.float32),
      pltpu.VMEM((SL,), jnp.float32),
      pltpu.VMEM((CH,), jnp.int32),
      pltpu.VMEM((CH,), jnp.int32),
      pltpu.VMEM((CH,), jnp.float32),
      pltpu.SemaphoreType.DMA,
  ]
  return pl.kernel(
      _prep_kernel,
      out_type=(jax.ShapeDtypeStruct((NREL * EPAD,), jnp.float32),
                jax.ShapeDtypeStruct((NC * NREL * DEGROWS,), jnp.float32)),
      mesh=mesh,
      scratch_types=scratch,
  )


def _agg_kernel(NB, D, t0, t1, t2, t3, src_hbm, dst_hbm, scale_hbm, out_hbm,
                acc, rows, dstv, locv, srcv, sclv, sem):
  c = lax.axis_index("c")
  s = lax.axis_index("s")
  w = s * NC + c
  tables = [t0, t1, t2, t3]
  BSZ = NPAD // NB
  ACC = BSZ + DUST
  ZR = ACC // NS
  FLR = BSZ // NS
  iota = lax.iota(jnp.int32, LANES)
  zeros16 = jnp.zeros((LANES,), jnp.float32)

  for b in range(NB):
    lo = b * BSZ
    def zrow(i, _):
      for j in range(D // LANES):
        rows[i, pl.ds(j * LANES, LANES)] = zeros16
      return 0
    lax.fori_loop(0, CH, zrow, 0)
    nfull, rem = ZR // CH, ZR % CH
    for k in range(nfull):
      pltpu.sync_copy(rows, acc.at[pl.ds(s * ZR + k * CH, CH)])
    if rem:
      pltpu.sync_copy(rows.at[pl.ds(0, rem)],
                      acc.at[pl.ds(s * ZR + nfull * CH, rem)])
    plsc.subcore_barrier()

    for r in range(NREL):
      table = tables[r]
      def chunk(ci, _):
        off = r * EPAD + w * ET + ci * CH
        pltpu.sync_copy(dst_hbm.at[pl.ds(off, CH)], dstv)
        pltpu.sync_copy(src_hbm.at[pl.ds(off, CH)], srcv)
        pltpu.sync_copy(scale_hbm.at[pl.ds(off, CH)], sclv)
        for k in range(CH // LANES):
          dv = dstv[pl.ds(k * LANES, LANES)]
          inb = (dv >= lo) & (dv < lo + BSZ)
          locv[pl.ds(k * LANES, LANES)] = jnp.where(
              inb, dv - lo, BSZ + k * LANES + iota)
        pltpu.async_copy(table.at[srcv], rows, sem).wait()
        def sgrp(g, _):
          sv16 = sclv[pl.ds(g * LANES, LANES)]
          for i in range(LANES):
            svi = jnp.full((LANES,), sv16[i], jnp.float32)
            row = g * LANES + i
            for j in range(D // LANES):
              rows[row, pl.ds(j * LANES, LANES)] = (
                  rows[row, pl.ds(j * LANES, LANES)] * svi)
          return 0
        lax.fori_loop(0, CH // LANES, sgrp, 0)
        pltpu.sync_copy(rows, acc.at[locv], add=True)
        return 0
      lax.fori_loop(0, NCH, chunk, 0)
    plsc.subcore_barrier()

    pltpu.sync_copy(acc.at[pl.ds(s * FLR, FLR)],
                    out_hbm.at[c, pl.ds(lo + s * FLR, FLR)])
    plsc.subcore_barrier()


def _make_agg(NB, D):
  mesh = plsc.VectorSubcoreMesh(core_axis_name="c", subcore_axis_name="s")
  BSZ = NPAD // NB
  scratch = [
      pltpu.VMEM_SHARED((BSZ + DUST, D), jnp.float32),
      pltpu.VMEM((CH, D), jnp.float32),
      pltpu.VMEM((CH,), jnp.int32),
      pltpu.VMEM((CH,), jnp.int32),
      pltpu.VMEM((CH,), jnp.int32),
      pltpu.VMEM((CH,), jnp.float32),
      pltpu.SemaphoreType.DMA,
  ]
  return pl.kernel(
      functools.partial(_agg_kernel, NB, D),
      out_type=jax.ShapeDtypeStruct((NC, NPAD, D), jnp.float32),
      mesh=mesh,
      scratch_types=scratch,
      compiler_params=pltpu.CompilerParams(use_tc_tiling_on_sc=(D == H)),
  )


def _tc_transform(p0, p1, bias, wc, basis):
  Dout = basis.shape[2]
  R = 512
  G = NPAD // R

  def body(p0_ref, p1_ref, b_ref, wc_ref, ba_ref, out_ref):
    x = p0_ref[...] + p1_ref[...] + b_ref[...]
    h = jnp.maximum(x, 0.0)
    wcv = wc_ref[...]
    for r in range(NREL):
      wr = wcv[r, 0] * ba_ref[0] + wcv[r, 1] * ba_ref[1]
      out_ref[r] = jnp.dot(h, wr, preferred_element_type=jnp.float32)

  return pl.pallas_call(
      body,
      grid=(G,),
      in_specs=[
          pl.BlockSpec((R, H), lambda i: (i, 0)),
          pl.BlockSpec((R, H), lambda i: (i, 0)),
          pl.BlockSpec((1, H), lambda i: (0, 0)),
          pl.BlockSpec((NREL, 2), lambda i: (0, 0)),
          pl.BlockSpec((2, H, Dout), lambda i: (0, 0, 0)),
      ],
      out_specs=pl.BlockSpec((NREL, R, Dout), lambda i: (0, i, 0)),
      out_shape=jax.ShapeDtypeStruct((NREL, NPAD, Dout), jnp.float32),
  )(p0, p1, bias.reshape(1, H), wc, basis)


def _tc_final(p0, p1, bias2):
  R = 400
  G = N // R

  def body(p0_ref, p1_ref, b_ref, out_ref):
    out_ref[...] = p0_ref[...] + p1_ref[...] + b_ref[...]

  return pl.pallas_call(
      body,
      grid=(G,),
      in_specs=[
          pl.BlockSpec((R, CLS), lambda i: (i, 0)),
          pl.BlockSpec((R, CLS), lambda i: (i, 0)),
          pl.BlockSpec((1, CLS), lambda i: (0, 0)),
      ],
      out_specs=pl.BlockSpec((R, CLS), lambda i: (i, 0)),
      out_shape=jax.ShapeDtypeStruct((N, CLS), jnp.float32),
  )(p0, p1, bias2.reshape(1, CLS))


def kernel(edge_index, embeds, embed_bias, weight1, w_comp1, bias1, weight2,
           w_comp2, bias2):
  src = edge_index[:, 0, :]
  dst = edge_index[:, 1, :]
  src_p = jnp.pad(src, ((0, 0), (0, EPAD - E)), constant_values=0).reshape(-1)
  dst_p = jnp.pad(dst, ((0, 0), (0, EPAD - E)),
                  constant_values=NPAD).reshape(-1)

  scale, _inv = _make_prep()(dst_p)

  agg128 = _make_agg(4, H)
  agg16 = _make_agg(1, CLS)

  pA = agg128(embeds[0], embeds[1], embeds[2], embeds[3],
              src_p, dst_p, scale)
  y1 = _tc_transform(pA[0], pA[1], embed_bias, w_comp1, weight1)
  pB = agg128(y1[0], y1[1], y1[2], y1[3], src_p, dst_p, scale)
  y2 = _tc_transform(pB[0], pB[1], bias1, w_comp2, weight2)
  pC = agg16(y2[0], y2[1], y2[2], y2[3], src_p, dst_p, scale)
  return _tc_final(pC[0], pC[1], bias2)
